# baseline (speedup 1.0000x reference)
import jax
import jax.numpy as jnp
from jax.experimental import pallas as pl
from jax.experimental.pallas import tpu as pltpu

NQ = 4
K = 1024


def _pq_body(x_ref, e_ref, qz_ref, idx_ref, en_ref, ehi_ref, elo_ref):
    @pl.when(pl.program_id(1) == 0)
    def _prep():
        e = e_ref[0]
        en_ref[...] = e / jnp.clip(
            jnp.sqrt(jnp.sum(e * e, axis=1, keepdims=True)), 1e-12)
        ehi = e.astype(jnp.bfloat16)
        ehi_ref[...] = ehi
        elo_ref[...] = (e - ehi.astype(jnp.float32)).astype(jnp.bfloat16)

    xb = x_ref[0, 0]
    xn = xb / jnp.clip(jnp.sqrt(jnp.sum(xb * xb, axis=0, keepdims=True)), 1e-12)
    dist_t = jax.lax.dot_general(
        en_ref[...], xn, (((1,), (0,)), ((), ())),
        preferred_element_type=jnp.float32)
    idx = jnp.argmax(dist_t, axis=0)
    idx_ref[0, 0, 0] = idx
    one_hot = (jax.lax.broadcasted_iota(jnp.int32, dist_t.shape, 0)
               == idx[None, :]).astype(jnp.float32).astype(jnp.bfloat16)
    qz_ref[0, 0] = (
        jax.lax.dot_general(ehi_ref[...], one_hot, (((0,), (0,)), ((), ())),
                            preferred_element_type=jnp.float32)
        + jax.lax.dot_general(elo_ref[...], one_hot, (((0,), (0,)), ((), ())),
                              preferred_element_type=jnp.float32))


def kernel(x, embed):
    B, C, H, W = x.shape
    nq, k, cq = embed.shape
    hw = H * W
    xg = x.reshape(B, nq, cq, hw)

    qz, idx = pl.pallas_call(
        _pq_body,
        grid=(nq, B),
        in_specs=[
            pl.BlockSpec((1, 1, cq, hw), lambda q, b: (b, q, 0, 0)),
            pl.BlockSpec((1, k, cq), lambda q, b: (q, 0, 0)),
        ],
        out_specs=[
            pl.BlockSpec((1, 1, cq, hw), lambda q, b: (b, q, 0, 0)),
            pl.BlockSpec((1, 1, 1, hw), lambda q, b: (b, q, 0, 0)),
        ],
        out_shape=[
            jax.ShapeDtypeStruct((B, nq, cq, hw), jnp.float32),
            jax.ShapeDtypeStruct((B, nq, 1, hw), jnp.int32),
        ],
        scratch_shapes=[
            pltpu.VMEM((k, cq), jnp.float32),
            pltpu.VMEM((k, cq), jnp.bfloat16),
            pltpu.VMEM((k, cq), jnp.bfloat16),
        ],
        compiler_params=pltpu.CompilerParams(
            dimension_semantics=("arbitrary", "arbitrary")),
    )(xg, embed)

    quantized = qz.reshape(B, C, H, W)
    encoding = idx.reshape(B, nq * H, W)
    vq_loss = jnp.zeros((1,), dtype=jnp.float32)
    return quantized, encoding, vq_loss


# speedup vs baseline: 1.1882x; 1.1882x over previous
import jax
import jax.numpy as jnp
from jax.experimental import pallas as pl
from jax.experimental.pallas import tpu as pltpu

NQ = 4
K = 1024


BB = 8


def _pq_body(x_ref, e_ref, qz_ref, idx_ref, en_ref):
    @pl.when(pl.program_id(1) == 0)
    def _prep():
        e = e_ref[0]
        en_ref[...] = e / jnp.clip(
            jnp.sqrt(jnp.sum(e * e, axis=1, keepdims=True)), 1e-12)

    e = e_ref[0]
    for i in range(BB):
        xb = x_ref[i, 0]
        xn = xb / jnp.clip(
            jnp.sqrt(jnp.sum(xb * xb, axis=0, keepdims=True)), 1e-12)
        dist_t = jax.lax.dot_general(
            en_ref[...], xn, (((1,), (0,)), ((), ())),
            preferred_element_type=jnp.float32)
        idx = jnp.argmax(dist_t, axis=0)
        idx_ref[i, 0, 0] = idx
        one_hot = (jax.lax.broadcasted_iota(jnp.int32, dist_t.shape, 0)
                   == idx[None, :]).astype(jnp.float32)
        qz_ref[i, 0] = jax.lax.dot_general(
            e, one_hot, (((0,), (0,)), ((), ())),
            preferred_element_type=jnp.float32)


def kernel(x, embed):
    B, C, H, W = x.shape
    nq, k, cq = embed.shape
    hw = H * W
    xg = x.reshape(B, nq, cq, hw)

    qz, idx = pl.pallas_call(
        _pq_body,
        grid=(nq, B // BB),
        in_specs=[
            pl.BlockSpec((BB, 1, cq, hw), lambda q, b: (b, q, 0, 0)),
            pl.BlockSpec((1, k, cq), lambda q, b: (q, 0, 0)),
        ],
        out_specs=[
            pl.BlockSpec((BB, 1, cq, hw), lambda q, b: (b, q, 0, 0)),
            pl.BlockSpec((BB, 1, 1, hw), lambda q, b: (b, q, 0, 0)),
        ],
        out_shape=[
            jax.ShapeDtypeStruct((B, nq, cq, hw), jnp.float32),
            jax.ShapeDtypeStruct((B, nq, 1, hw), jnp.int32),
        ],
        scratch_shapes=[
            pltpu.VMEM((k, cq), jnp.float32),
        ],
        compiler_params=pltpu.CompilerParams(
            dimension_semantics=("arbitrary", "arbitrary")),
    )(xg, embed)

    quantized = qz.reshape(B, C, H, W)
    encoding = idx.reshape(B, nq * H, W)
    vq_loss = jnp.zeros((1,), dtype=jnp.float32)
    return quantized, encoding, vq_loss
